# Initial kernel scaffold; baseline (speedup 1.0000x reference)
import jax
import jax.numpy as jnp
from jax.experimental import pallas as pl


def kernel(h_node, adjacency, W_msg, b_msg, W_ih, W_hh, b_ih, b_hh):
    raise NotImplementedError("write your pallas kernel here")


# speedup vs baseline: 1.2870x; 1.2870x over previous
import functools

import jax
import jax.numpy as jnp
from jax.experimental import pallas as pl
from jax.experimental.pallas import tpu as pltpu

_PASSES = 3


def _ggnn_body(h0_ref, adj_ref, wmsg_ref, bmsg_ref, wih_ref, whh_ref,
               bih_ref, bhh_ref, out_ref, msg_ref, h_ref, *, T, BR, R, D, M):
    p = pl.program_id(0)
    r = pl.program_id(1)

    @pl.when(jnp.logical_and(p == 0, r == 0))
    def _init():
        h_ref[...] = h0_ref[...]

    @pl.when(r == 0)
    def _messages():
        h_cur = h_ref[...]
        for i in range(T):
            msg_ref[i] = (
                jnp.dot(h_cur, wmsg_ref[i], preferred_element_type=jnp.float32)
                + bmsg_ref[i]
            )

    rows = pl.ds(r * BR, BR)
    acc = jnp.zeros((BR, M), dtype=jnp.float32)
    for i in range(T):
        acc = acc + jnp.dot(adj_ref[i], msg_ref[i],
                            preferred_element_type=jnp.float32)
    x = jnp.maximum(acc, 0.0)

    h = h_ref[rows, :]
    gi = jnp.dot(x, wih_ref[...], preferred_element_type=jnp.float32) + bih_ref[...]
    gh = jnp.dot(h, whh_ref[...], preferred_element_type=jnp.float32) + bhh_ref[...]
    i_r, i_z, i_n = gi[:, :D], gi[:, D:2 * D], gi[:, 2 * D:]
    h_r, h_z, h_n = gh[:, :D], gh[:, D:2 * D], gh[:, 2 * D:]
    rg = jax.nn.sigmoid(i_r + h_r)
    zg = jax.nn.sigmoid(i_z + h_z)
    ng = jnp.tanh(i_n + rg * h_n)
    h_new = (1.0 - zg) * ng + zg * h
    h_ref[rows, :] = h_new
    out_ref[rows, :] = h_new


def kernel(h_node, adjacency, W_msg, b_msg, W_ih, W_hh, b_ih, b_hh):
    N, D = h_node.shape
    T = adjacency.shape[0]
    M = W_msg.shape[1]
    G = 3 * D
    BR = 128 if N % 128 == 0 else N
    R = N // BR

    grid = (_PASSES, R)
    in_specs = [
        pl.BlockSpec((N, D), lambda p, r: (0, 0)),
        pl.BlockSpec((T, BR, N), lambda p, r: (0, r, 0)),
        pl.BlockSpec((T, D, M), lambda p, r: (0, 0, 0)),
        pl.BlockSpec((T, 1, M), lambda p, r: (0, 0, 0)),
        pl.BlockSpec((M, G), lambda p, r: (0, 0)),
        pl.BlockSpec((D, G), lambda p, r: (0, 0)),
        pl.BlockSpec((1, G), lambda p, r: (0, 0)),
        pl.BlockSpec((1, G), lambda p, r: (0, 0)),
    ]
    out_specs = pl.BlockSpec((N, D), lambda p, r: (0, 0))
    scratch_shapes = [
        pltpu.VMEM((T, N, M), jnp.float32),
        pltpu.VMEM((N, D), jnp.float32),
    ]

    f = pl.pallas_call(
        functools.partial(_ggnn_body, T=T, BR=BR, R=R, D=D, M=M),
        grid=grid,
        in_specs=in_specs,
        out_specs=out_specs,
        out_shape=jax.ShapeDtypeStruct((N, D), jnp.float32),
        scratch_shapes=scratch_shapes,
        compiler_params=pltpu.CompilerParams(
            dimension_semantics=("arbitrary", "arbitrary")),
    )
    return f(h_node, adjacency, jnp.transpose(W_msg, (0, 2, 1)),
             b_msg.reshape(T, 1, M), W_ih.T, W_hh.T,
             b_ih.reshape(1, G), b_hh.reshape(1, G))
